# baseline (speedup 1.0000x reference)
import jax
import jax.numpy as jnp
from jax import lax
from jax.experimental import pallas as pl
from jax.experimental.pallas import tpu as pltpu

C = 256
T = 10000
THRESH = 0.6
H = 2000
NCHUNK = T // H


NBUF = 3
NSPLIT = 5
HSUB = H // NSPLIT


def _fused(d_hbm, wg_ref, bg_ref, wc_ref, bc_ref, out_ref, *scratch):
    bufs = scratch[:NBUF]
    sems = scratch[NBUF:]

    def copies(i):
        b = i % NBUF
        return [
            pltpu.make_async_copy(
                d_hbm.at[pl.ds(i * H + k * HSUB, HSUB), :],
                bufs[b].at[pl.ds(k * HSUB, HSUB), :],
                sems[b * NSPLIT + k])
            for k in range(NSPLIT)
        ]

    for cp in copies(0):
        cp.start()
    for cp in copies(1):
        cp.start()

    g = g3 = g4 = None
    s1 = None
    dims = (((0,), (0,)), ((), ()))
    for i in range(NCHUNK):
        if i + 2 < NCHUNK:
            for cp in copies(i + 2):
                cp.start()
        for cp in copies(i):
            cp.wait()
        d = bufs[i % NBUF][...]
        p1 = jnp.sum(d, axis=0, keepdims=True)
        db = d.astype(jnp.bfloat16)
        db2 = db * db
        pg = lax.dot_general(db, db, dims, preferred_element_type=jnp.float32)
        pg3 = lax.dot_general(db, db2, dims, preferred_element_type=jnp.float32)
        pg4 = lax.dot_general(db2, db2, dims, preferred_element_type=jnp.float32)
        if i == 0:
            g, g3, g4, s1 = pg, pg3, pg4, p1
        else:
            g, g3, g4, s1 = g + pg, g3 + pg3, g4 + pg4, s1 + p1

    inv_t = jnp.float32(1.0 / T)
    row = lax.broadcasted_iota(jnp.int32, (C, C), 0)
    colc = lax.broadcasted_iota(jnp.int32, (C, C), 1)
    on_diag = row == colc

    s2 = jnp.sum(jnp.where(on_diag, g, 0.0), axis=0, keepdims=True)
    s3 = jnp.sum(jnp.where(on_diag, g3, 0.0), axis=0, keepdims=True)
    s4 = jnp.sum(jnp.where(on_diag, g4, 0.0), axis=0, keepdims=True)

    mu = s1 * inv_t
    mu_c = jnp.transpose(mu)

    normsq = jnp.maximum(s2 - (jnp.float32(T) * mu) * mu, 0.0)
    inv_norm = lax.rsqrt(jnp.maximum(normsq, jnp.float32(1e-12)))
    corr = ((g - (jnp.float32(T) * mu_c) * mu)
            * inv_norm * jnp.transpose(inv_norm))
    corr = jnp.clip(corr, -1.0, 1.0)

    ac = jnp.abs(corr)
    mask = (ac >= jnp.float32(THRESH)) & (~on_diag)
    w = jnp.clip(ac, 1e-6, 0.99)
    a = jnp.where(mask, w, 0.0) + jnp.where(on_diag, 1.0, 0.0)

    m2 = s2 * inv_t - mu * mu
    m3 = s3 * inv_t - 3.0 * mu * (s2 * inv_t) + 2.0 * mu * mu * mu
    m4 = (s4 * inv_t - 4.0 * mu * (s3 * inv_t)
          + 6.0 * (mu * mu) * (s2 * inv_t) - 3.0 * (mu * mu) * (mu * mu))
    m2s = jnp.maximum(m2, jnp.float32(1e-12))
    inv_m2s = lax.rsqrt(m2s)
    skew = m3 * inv_m2s * inv_m2s * inv_m2s
    kurt = m4 * (inv_m2s * inv_m2s) * (inv_m2s * inv_m2s) - 3.0
    xt = jnp.concatenate([mu, m2, skew, kurt], axis=0)

    aggt = jnp.dot(xt, a, preferred_element_type=jnp.float32)
    ht = jnp.dot(wg_ref[...], aggt, preferred_element_type=jnp.float32)
    ht = jnp.maximum(ht + bg_ref[...], 0.0)
    get = jnp.sum(ht, axis=1, keepdims=True)
    ge = jnp.transpose(get)
    logits = jnp.dot(ge, wc_ref[...], preferred_element_type=jnp.float32)
    out_ref[...] = logits + bc_ref[...]


def kernel(data, W_gfc, b_gfc, W_cls, b_cls):
    dt = data.T
    out = pl.pallas_call(
        _fused,
        in_specs=[
            pl.BlockSpec(memory_space=pltpu.MemorySpace.HBM),
            pl.BlockSpec(memory_space=pltpu.MemorySpace.VMEM),
            pl.BlockSpec(memory_space=pltpu.MemorySpace.VMEM),
            pl.BlockSpec(memory_space=pltpu.MemorySpace.VMEM),
            pl.BlockSpec(memory_space=pltpu.MemorySpace.VMEM),
        ],
        out_specs=pl.BlockSpec(memory_space=pltpu.MemorySpace.VMEM),
        out_shape=jax.ShapeDtypeStruct((1, 2), jnp.float32),
        scratch_shapes=(
            [pltpu.VMEM((H, C), jnp.float32) for _ in range(NBUF)]
            + [pltpu.SemaphoreType.DMA for _ in range(NBUF * NSPLIT)]
        ),
    )(dt, W_gfc.T, b_gfc.reshape(-1, 1), W_cls, b_cls.reshape(1, -1))
    return out


# speedup vs baseline: 1.2179x; 1.2179x over previous
import jax
import jax.numpy as jnp
from jax import lax
from jax.experimental import pallas as pl
from jax.experimental.pallas import tpu as pltpu

C = 256
T = 10000
THRESH = 0.6
H = 2000
NCHUNK = T // H


NBUF = 3
NSPLIT = 2
HSUB = H // NSPLIT


def _fused(d_hbm, wg_ref, bg_ref, wc_ref, bc_ref, out_ref, *scratch):
    bufs = scratch[:NBUF]
    sems = scratch[NBUF:]

    def copies(i):
        b = i % NBUF
        return [
            pltpu.make_async_copy(
                d_hbm.at[pl.ds(i * H + k * HSUB, HSUB), :],
                bufs[b].at[pl.ds(k * HSUB, HSUB), :],
                sems[b * NSPLIT + k])
            for k in range(NSPLIT)
        ]

    for cp in copies(0):
        cp.start()
    for cp in copies(1):
        cp.start()

    g = None
    s1 = s2 = s3 = s4 = None
    dims = (((0,), (0,)), ((), ()))
    for i in range(NCHUNK):
        if i + 2 < NCHUNK:
            for cp in copies(i + 2):
                cp.start()
        for cp in copies(i):
            cp.wait()
        d = bufs[i % NBUF][...]
        d2 = d * d
        p1 = jnp.sum(d, axis=0, keepdims=True)
        p2 = jnp.sum(d2, axis=0, keepdims=True)
        p3 = jnp.sum(d2 * d, axis=0, keepdims=True)
        p4 = jnp.sum(d2 * d2, axis=0, keepdims=True)
        db = d.astype(jnp.bfloat16)
        pg = lax.dot_general(db, db, dims, preferred_element_type=jnp.float32)
        if i == 0:
            g, s1, s2, s3, s4 = pg, p1, p2, p3, p4
        else:
            g, s1, s2, s3, s4 = g + pg, s1 + p1, s2 + p2, s3 + p3, s4 + p4

    inv_t = jnp.float32(1.0 / T)
    row = lax.broadcasted_iota(jnp.int32, (C, C), 0)
    colc = lax.broadcasted_iota(jnp.int32, (C, C), 1)
    on_diag = row == colc

    mu = s1 * inv_t
    mu_c = jnp.transpose(mu)

    normsq = jnp.maximum(s2 - (jnp.float32(T) * mu) * mu, 0.0)
    inv_norm = lax.rsqrt(jnp.maximum(normsq, jnp.float32(1e-12)))
    corr = ((g - (jnp.float32(T) * mu_c) * mu)
            * inv_norm * jnp.transpose(inv_norm))
    corr = jnp.clip(corr, -1.0, 1.0)

    ac = jnp.abs(corr)
    mask = (ac >= jnp.float32(THRESH)) & (~on_diag)
    w = jnp.clip(ac, 1e-6, 0.99)
    a = jnp.where(mask, w, 0.0) + jnp.where(on_diag, 1.0, 0.0)

    m2 = s2 * inv_t - mu * mu
    m3 = s3 * inv_t - 3.0 * mu * (s2 * inv_t) + 2.0 * mu * mu * mu
    m4 = (s4 * inv_t - 4.0 * mu * (s3 * inv_t)
          + 6.0 * (mu * mu) * (s2 * inv_t) - 3.0 * (mu * mu) * (mu * mu))
    m2s = jnp.maximum(m2, jnp.float32(1e-12))
    inv_m2s = lax.rsqrt(m2s)
    skew = m3 * inv_m2s * inv_m2s * inv_m2s
    kurt = m4 * (inv_m2s * inv_m2s) * (inv_m2s * inv_m2s) - 3.0
    xt = jnp.concatenate([mu, m2, skew, kurt], axis=0)

    aggt = jnp.dot(xt, a, preferred_element_type=jnp.float32)
    ht = jnp.dot(wg_ref[...], aggt, preferred_element_type=jnp.float32)
    ht = jnp.maximum(ht + bg_ref[...], 0.0)
    get = jnp.sum(ht, axis=1, keepdims=True)
    ge = jnp.transpose(get)
    logits = jnp.dot(ge, wc_ref[...], preferred_element_type=jnp.float32)
    out_ref[...] = logits + bc_ref[...]


def kernel(data, W_gfc, b_gfc, W_cls, b_cls):
    dt = data.T
    out = pl.pallas_call(
        _fused,
        in_specs=[
            pl.BlockSpec(memory_space=pltpu.MemorySpace.HBM),
            pl.BlockSpec(memory_space=pltpu.MemorySpace.VMEM),
            pl.BlockSpec(memory_space=pltpu.MemorySpace.VMEM),
            pl.BlockSpec(memory_space=pltpu.MemorySpace.VMEM),
            pl.BlockSpec(memory_space=pltpu.MemorySpace.VMEM),
        ],
        out_specs=pl.BlockSpec(memory_space=pltpu.MemorySpace.VMEM),
        out_shape=jax.ShapeDtypeStruct((1, 2), jnp.float32),
        scratch_shapes=(
            [pltpu.VMEM((H, C), jnp.float32) for _ in range(NBUF)]
            + [pltpu.SemaphoreType.DMA for _ in range(NBUF * NSPLIT)]
        ),
    )(dt, W_gfc.T, b_gfc.reshape(-1, 1), W_cls, b_cls.reshape(1, -1))
    return out


# speedup vs baseline: 1.2349x; 1.0139x over previous
import jax
import jax.numpy as jnp
from jax import lax
from jax.experimental import pallas as pl
from jax.experimental.pallas import tpu as pltpu

C = 256
T = 10000
THRESH = 0.6
H = 1000
NCHUNK = T // H


NBUF = 3
NSPLIT = 1
HSUB = H // NSPLIT


def _fused(d_hbm, wg_ref, bg_ref, wc_ref, bc_ref, out_ref, *scratch):
    bufs = scratch[:NBUF]
    sems = scratch[NBUF:]

    def copies(i):
        b = i % NBUF
        return [
            pltpu.make_async_copy(
                d_hbm.at[pl.ds(i * H + k * HSUB, HSUB), :],
                bufs[b].at[pl.ds(k * HSUB, HSUB), :],
                sems[b * NSPLIT + k])
            for k in range(NSPLIT)
        ]

    for cp in copies(0):
        cp.start()
    for cp in copies(1):
        cp.start()

    g = None
    s1 = s2 = s3 = s4 = None
    dims = (((0,), (0,)), ((), ()))
    for i in range(NCHUNK):
        if i + 2 < NCHUNK:
            for cp in copies(i + 2):
                cp.start()
        for cp in copies(i):
            cp.wait()
        d = bufs[i % NBUF][...]
        d2 = d * d
        p1 = jnp.sum(d, axis=0, keepdims=True)
        p2 = jnp.sum(d2, axis=0, keepdims=True)
        p3 = jnp.sum(d2 * d, axis=0, keepdims=True)
        p4 = jnp.sum(d2 * d2, axis=0, keepdims=True)
        db = d.astype(jnp.bfloat16)
        pg = lax.dot_general(db, db, dims, preferred_element_type=jnp.float32)
        if i == 0:
            g, s1, s2, s3, s4 = pg, p1, p2, p3, p4
        else:
            g, s1, s2, s3, s4 = g + pg, s1 + p1, s2 + p2, s3 + p3, s4 + p4

    inv_t = jnp.float32(1.0 / T)
    row = lax.broadcasted_iota(jnp.int32, (C, C), 0)
    colc = lax.broadcasted_iota(jnp.int32, (C, C), 1)
    on_diag = row == colc

    mu = s1 * inv_t
    mu_c = jnp.transpose(mu)

    normsq = jnp.maximum(s2 - (jnp.float32(T) * mu) * mu, 0.0)
    inv_norm = lax.rsqrt(jnp.maximum(normsq, jnp.float32(1e-12)))
    corr = ((g - (jnp.float32(T) * mu_c) * mu)
            * inv_norm * jnp.transpose(inv_norm))
    corr = jnp.clip(corr, -1.0, 1.0)

    ac = jnp.abs(corr)
    mask = (ac >= jnp.float32(THRESH)) & (~on_diag)
    w = jnp.clip(ac, 1e-6, 0.99)
    a = jnp.where(mask, w, 0.0) + jnp.where(on_diag, 1.0, 0.0)

    m2 = s2 * inv_t - mu * mu
    m3 = s3 * inv_t - 3.0 * mu * (s2 * inv_t) + 2.0 * mu * mu * mu
    m4 = (s4 * inv_t - 4.0 * mu * (s3 * inv_t)
          + 6.0 * (mu * mu) * (s2 * inv_t) - 3.0 * (mu * mu) * (mu * mu))
    m2s = jnp.maximum(m2, jnp.float32(1e-12))
    inv_m2s = lax.rsqrt(m2s)
    skew = m3 * inv_m2s * inv_m2s * inv_m2s
    kurt = m4 * (inv_m2s * inv_m2s) * (inv_m2s * inv_m2s) - 3.0
    xt = jnp.concatenate([mu, m2, skew, kurt], axis=0)

    aggt = jnp.dot(xt, a, preferred_element_type=jnp.float32)
    ht = jnp.dot(wg_ref[...], aggt, preferred_element_type=jnp.float32)
    ht = jnp.maximum(ht + bg_ref[...], 0.0)
    get = jnp.sum(ht, axis=1, keepdims=True)
    ge = jnp.transpose(get)
    logits = jnp.dot(ge, wc_ref[...], preferred_element_type=jnp.float32)
    out_ref[...] = logits + bc_ref[...]


def kernel(data, W_gfc, b_gfc, W_cls, b_cls):
    dt = data.T
    out = pl.pallas_call(
        _fused,
        in_specs=[
            pl.BlockSpec(memory_space=pltpu.MemorySpace.HBM),
            pl.BlockSpec(memory_space=pltpu.MemorySpace.VMEM),
            pl.BlockSpec(memory_space=pltpu.MemorySpace.VMEM),
            pl.BlockSpec(memory_space=pltpu.MemorySpace.VMEM),
            pl.BlockSpec(memory_space=pltpu.MemorySpace.VMEM),
        ],
        out_specs=pl.BlockSpec(memory_space=pltpu.MemorySpace.VMEM),
        out_shape=jax.ShapeDtypeStruct((1, 2), jnp.float32),
        scratch_shapes=(
            [pltpu.VMEM((H, C), jnp.float32) for _ in range(NBUF)]
            + [pltpu.SemaphoreType.DMA for _ in range(NBUF * NSPLIT)]
        ),
    )(dt, W_gfc.T, b_gfc.reshape(-1, 1), W_cls, b_cls.reshape(1, -1))
    return out


# speedup vs baseline: 1.3085x; 1.0596x over previous
import jax
import jax.numpy as jnp
from jax import lax
from jax.experimental import pallas as pl
from jax.experimental.pallas import tpu as pltpu

C = 256
T = 10000
THRESH = 0.6
H = 2000
NCHUNK = T // H


NBUF = 3
NSPLIT = 1
HSUB = H // NSPLIT


def _fused(d_hbm, wg_ref, bg_ref, wc_ref, bc_ref, out_ref, *scratch):
    bufs = scratch[:NBUF]
    sems = scratch[NBUF:]

    def copies(i):
        b = i % NBUF
        return [
            pltpu.make_async_copy(
                d_hbm.at[pl.ds(i * H + k * HSUB, HSUB), :],
                bufs[b].at[pl.ds(k * HSUB, HSUB), :],
                sems[b * NSPLIT + k])
            for k in range(NSPLIT)
        ]

    for cp in copies(0):
        cp.start()
    for cp in copies(1):
        cp.start()

    g = None
    s1 = s2 = s3 = s4 = None
    dims = (((0,), (0,)), ((), ()))
    for i in range(NCHUNK):
        if i + 2 < NCHUNK:
            for cp in copies(i + 2):
                cp.start()
        for cp in copies(i):
            cp.wait()
        d = bufs[i % NBUF][...]
        d2 = d * d
        p1 = jnp.sum(d, axis=0, keepdims=True)
        p2 = jnp.sum(d2, axis=0, keepdims=True)
        p3 = jnp.sum(d2 * d, axis=0, keepdims=True)
        p4 = jnp.sum(d2 * d2, axis=0, keepdims=True)
        db = d.astype(jnp.bfloat16)
        pg = lax.dot_general(db, db, dims, preferred_element_type=jnp.float32)
        if i == 0:
            g, s1, s2, s3, s4 = pg, p1, p2, p3, p4
        else:
            g, s1, s2, s3, s4 = g + pg, s1 + p1, s2 + p2, s3 + p3, s4 + p4

    inv_t = jnp.float32(1.0 / T)
    row = lax.broadcasted_iota(jnp.int32, (C, C), 0)
    colc = lax.broadcasted_iota(jnp.int32, (C, C), 1)
    on_diag = row == colc

    mu = s1 * inv_t
    mu_c = jnp.transpose(mu)

    normsq = jnp.maximum(s2 - (jnp.float32(T) * mu) * mu, 0.0)
    inv_norm = lax.rsqrt(jnp.maximum(normsq, jnp.float32(1e-12)))
    corr = ((g - (jnp.float32(T) * mu_c) * mu)
            * inv_norm * jnp.transpose(inv_norm))
    corr = jnp.clip(corr, -1.0, 1.0)

    ac = jnp.abs(corr)
    mask = (ac >= jnp.float32(THRESH)) & (~on_diag)
    w = jnp.clip(ac, 1e-6, 0.99)
    a = jnp.where(mask, w, 0.0) + jnp.where(on_diag, 1.0, 0.0)

    m2 = s2 * inv_t - mu * mu
    m3 = s3 * inv_t - 3.0 * mu * (s2 * inv_t) + 2.0 * mu * mu * mu
    m4 = (s4 * inv_t - 4.0 * mu * (s3 * inv_t)
          + 6.0 * (mu * mu) * (s2 * inv_t) - 3.0 * (mu * mu) * (mu * mu))
    m2s = jnp.maximum(m2, jnp.float32(1e-12))
    inv_m2s = lax.rsqrt(m2s)
    skew = m3 * inv_m2s * inv_m2s * inv_m2s
    kurt = m4 * (inv_m2s * inv_m2s) * (inv_m2s * inv_m2s) - 3.0
    xt = jnp.concatenate([mu, m2, skew, kurt], axis=0)

    aggt = jnp.dot(xt, a, preferred_element_type=jnp.float32)
    ht = jnp.dot(wg_ref[...], aggt, preferred_element_type=jnp.float32)
    ht = jnp.maximum(ht + bg_ref[...], 0.0)
    get = jnp.sum(ht, axis=1, keepdims=True)
    ge = jnp.transpose(get)
    logits = jnp.dot(ge, wc_ref[...], preferred_element_type=jnp.float32)
    out_ref[...] = logits + bc_ref[...]


def kernel(data, W_gfc, b_gfc, W_cls, b_cls):
    dt = data.T
    out = pl.pallas_call(
        _fused,
        in_specs=[
            pl.BlockSpec(memory_space=pltpu.MemorySpace.HBM),
            pl.BlockSpec(memory_space=pltpu.MemorySpace.VMEM),
            pl.BlockSpec(memory_space=pltpu.MemorySpace.VMEM),
            pl.BlockSpec(memory_space=pltpu.MemorySpace.VMEM),
            pl.BlockSpec(memory_space=pltpu.MemorySpace.VMEM),
        ],
        out_specs=pl.BlockSpec(memory_space=pltpu.MemorySpace.VMEM),
        out_shape=jax.ShapeDtypeStruct((1, 2), jnp.float32),
        scratch_shapes=(
            [pltpu.VMEM((H, C), jnp.float32) for _ in range(NBUF)]
            + [pltpu.SemaphoreType.DMA for _ in range(NBUF * NSPLIT)]
        ),
    )(dt, W_gfc.T, b_gfc.reshape(-1, 1), W_cls, b_cls.reshape(1, -1))
    return out
